# baseline (device time: 249060 ns/iter reference)
import jax
import jax.numpy as jnp
from jax import lax
from jax.experimental import pallas as pl
from jax.experimental.pallas import tpu as pltpu

N_DEV = 32


def kernel(x, w_mat, scale_x, scale_w):
    m_per, k = x.shape
    n_per = w_mat.shape[1]

    def body(x_ref, w_ref, sx_ref, sw_ref, out_ref, xfull_ref, send_sems, recv_sems):
        my = lax.axis_index("i")
        left = (my - 1 + N_DEV) % N_DEV
        right = (my + 1) % N_DEV

        barrier_sem = pltpu.get_barrier_semaphore()
        for nbr in (left, right):
            pl.semaphore_signal(
                barrier_sem, inc=1,
                device_id=(nbr,), device_id_type=pl.DeviceIdType.MESH,
            )
        pl.semaphore_wait(barrier_sem, 2)

        xfull_ref[pl.ds(my * m_per, m_per), :] = x_ref[...]

        for h in range(N_DEV - 1):
            origin_send = (my - h + N_DEV) % N_DEV
            rdma = pltpu.make_async_remote_copy(
                src_ref=xfull_ref.at[pl.ds(origin_send * m_per, m_per), :],
                dst_ref=xfull_ref.at[pl.ds(origin_send * m_per, m_per), :],
                send_sem=send_sems.at[h],
                recv_sem=recv_sems.at[h],
                device_id=(right,),
                device_id_type=pl.DeviceIdType.MESH,
            )
            rdma.start()
            rdma.wait()

        acc = lax.dot_general(
            xfull_ref[...], w_ref[...],
            (((1,), (0,)), ((), ())),
            preferred_element_type=jnp.int32,
        )
        scale = sx_ref[0] * sw_ref[0]
        out_ref[...] = jnp.maximum(acc.astype(jnp.float32) * scale, 0.0)

    return pl.pallas_call(
        body,
        out_shape=jax.ShapeDtypeStruct((N_DEV * m_per, n_per), jnp.float32),
        in_specs=[
            pl.BlockSpec(memory_space=pltpu.VMEM),
            pl.BlockSpec(memory_space=pltpu.VMEM),
            pl.BlockSpec(memory_space=pltpu.SMEM),
            pl.BlockSpec(memory_space=pltpu.SMEM),
        ],
        out_specs=pl.BlockSpec(memory_space=pltpu.VMEM),
        scratch_shapes=[
            pltpu.VMEM((N_DEV * m_per, k), jnp.int8),
            pltpu.SemaphoreType.DMA((N_DEV - 1,)),
            pltpu.SemaphoreType.DMA((N_DEV - 1,)),
        ],
        compiler_params=pltpu.CompilerParams(collective_id=0),
    )(x, w_mat, scale_x, scale_w)


# device time: 143675 ns/iter; 1.7335x vs baseline; 1.7335x over previous
import numpy as np

import jax
import jax.numpy as jnp
from jax import lax
from jax.experimental import pallas as pl
from jax.experimental.pallas import tpu as pltpu

N_DEV = 32
R_HOPS = 16
L_HOPS = 15

_ring_cache = None


def _hamiltonian_ring():
    global _ring_cache
    if _ring_cache is not None:
        return _ring_cache
    import distributed_mesh_v7x as dm

    mesh = dm.get_mesh("i", world_size=N_DEV)
    coords = [tuple(d.coords) for d in mesh.devices.flat]
    coord_to_logical = {c: j for j, c in enumerate(coords)}
    p = [(0, 0), (1, 0), (2, 0), (3, 0), (3, 1), (2, 1), (1, 1), (0, 1),
         (0, 2), (1, 2), (2, 2), (3, 2), (3, 3), (2, 3), (1, 3), (0, 3)]
    cycle = [(0, y, z) for (y, z) in p] + [(1, y, z) for (y, z) in reversed(p)]
    ring = np.array([coord_to_logical[c] for c in cycle], dtype=np.int32)
    inv = np.empty(N_DEV, dtype=np.int32)
    inv[ring] = np.arange(N_DEV, dtype=np.int32)
    _ring_cache = (ring, inv)
    return _ring_cache


def kernel(x, w_mat, scale_x, scale_w):
    m_per, k = x.shape
    n_per = w_mat.shape[1]

    ring_np, inv_np = _hamiltonian_ring()
    ring = jnp.asarray(ring_np)
    my = lax.axis_index("i")
    r = jnp.asarray(inv_np)[my]
    right = ring[(r + 1) % N_DEV]
    left = ring[(r - 1) % N_DEV]

    hr = jnp.arange(R_HOPS)
    hl = jnp.arange(L_HOPS)
    send_r = ring[(r - hr) % N_DEV]
    recv_r = ring[(r - 1 - hr) % N_DEV]
    send_l = ring[(r + hl) % N_DEV]
    recv_l = ring[(r + 1 + hl) % N_DEV]

    meta = jnp.concatenate(
        [right[None], left[None], send_r, recv_r, send_l, recv_l]
    ).astype(jnp.int32)

    def body(meta_ref, x_ref, w_ref, sx_ref, sw_ref, out_ref,
             xfull_ref, sems_sr, sems_rr, sems_sl, sems_rl):
        rt = meta_ref[0]
        lt = meta_ref[1]

        barrier_sem = pltpu.get_barrier_semaphore()
        for nbr in (lt, rt):
            pl.semaphore_signal(
                barrier_sem, inc=1,
                device_id=(nbr,), device_id_type=pl.DeviceIdType.MESH,
            )
        pl.semaphore_wait(barrier_sem, 2)

        my_rows = meta_ref[2] * m_per
        xfull_ref[pl.ds(my_rows, m_per), :] = x_ref[...]

        def slab(origin):
            return xfull_ref.at[pl.ds(origin * m_per, m_per), :]

        senders = []
        for h in range(R_HOPS):
            o_sr = meta_ref[2 + h]
            d_r = pltpu.make_async_remote_copy(
                src_ref=slab(o_sr), dst_ref=slab(o_sr),
                send_sem=sems_sr.at[h], recv_sem=sems_rr.at[h],
                device_id=(rt,), device_id_type=pl.DeviceIdType.MESH,
            )
            d_r.start()
            senders.append(d_r)
            if h < L_HOPS:
                o_sl = meta_ref[34 + h]
                d_l = pltpu.make_async_remote_copy(
                    src_ref=slab(o_sl), dst_ref=slab(o_sl),
                    send_sem=sems_sl.at[h], recv_sem=sems_rl.at[h],
                    device_id=(lt,), device_id_type=pl.DeviceIdType.MESH,
                )
                d_l.start()
                senders.append(d_l)

            o_rr = meta_ref[18 + h]
            pltpu.make_async_remote_copy(
                src_ref=slab(o_rr), dst_ref=slab(o_rr),
                send_sem=sems_sr.at[h], recv_sem=sems_rr.at[h],
                device_id=(rt,), device_id_type=pl.DeviceIdType.MESH,
            ).wait_recv()
            if h < L_HOPS:
                o_rl = meta_ref[49 + h]
                pltpu.make_async_remote_copy(
                    src_ref=slab(o_rl), dst_ref=slab(o_rl),
                    send_sem=sems_sl.at[h], recv_sem=sems_rl.at[h],
                    device_id=(lt,), device_id_type=pl.DeviceIdType.MESH,
                ).wait_recv()

        for d in senders:
            d.wait_send()

        acc = lax.dot_general(
            xfull_ref[...], w_ref[...],
            (((1,), (0,)), ((), ())),
            preferred_element_type=jnp.int32,
        )
        scale = sx_ref[0] * sw_ref[0]
        out_ref[...] = jnp.maximum(acc.astype(jnp.float32) * scale, 0.0)

    return pl.pallas_call(
        body,
        out_shape=jax.ShapeDtypeStruct((N_DEV * m_per, n_per), jnp.float32),
        in_specs=[
            pl.BlockSpec(memory_space=pltpu.SMEM),
            pl.BlockSpec(memory_space=pltpu.VMEM),
            pl.BlockSpec(memory_space=pltpu.VMEM),
            pl.BlockSpec(memory_space=pltpu.SMEM),
            pl.BlockSpec(memory_space=pltpu.SMEM),
        ],
        out_specs=pl.BlockSpec(memory_space=pltpu.VMEM),
        scratch_shapes=[
            pltpu.VMEM((N_DEV * m_per, k), jnp.int8),
            pltpu.SemaphoreType.DMA((R_HOPS,)),
            pltpu.SemaphoreType.DMA((R_HOPS,)),
            pltpu.SemaphoreType.DMA((L_HOPS,)),
            pltpu.SemaphoreType.DMA((L_HOPS,)),
        ],
        compiler_params=pltpu.CompilerParams(collective_id=0),
    )(meta, x, w_mat, scale_x, scale_w)


# device time: 117363 ns/iter; 2.1221x vs baseline; 1.2242x over previous
import numpy as np

import jax
import jax.numpy as jnp
from jax import lax
from jax.experimental import pallas as pl
from jax.experimental.pallas import tpu as pltpu

N_DEV = 32
R_HOPS = 16
L_HOPS = 15
SUBS = 4

_ring_cache = None


def _hamiltonian_ring():
    global _ring_cache
    if _ring_cache is not None:
        return _ring_cache
    import distributed_mesh_v7x as dm

    mesh = dm.get_mesh("i", world_size=N_DEV)
    coords = [tuple(d.coords) for d in mesh.devices.flat]
    coord_to_logical = {c: j for j, c in enumerate(coords)}
    p = [(0, 0), (1, 0), (2, 0), (3, 0), (3, 1), (2, 1), (1, 1), (0, 1),
         (0, 2), (1, 2), (2, 2), (3, 2), (3, 3), (2, 3), (1, 3), (0, 3)]
    cycle = [(0, y, z) for (y, z) in p] + [(1, y, z) for (y, z) in reversed(p)]
    ring = np.array([coord_to_logical[c] for c in cycle], dtype=np.int32)
    inv = np.empty(N_DEV, dtype=np.int32)
    inv[ring] = np.arange(N_DEV, dtype=np.int32)
    _ring_cache = (ring, inv)
    return _ring_cache


def kernel(x, w_mat, scale_x, scale_w):
    m_per, k = x.shape
    n_per = w_mat.shape[1]
    sub_m = m_per // SUBS

    ring_np, inv_np = _hamiltonian_ring()
    ring = jnp.asarray(ring_np)
    my = lax.axis_index("i")
    r = jnp.asarray(inv_np)[my]
    right = ring[(r + 1) % N_DEV]
    left = ring[(r - 1) % N_DEV]

    hr = jnp.arange(R_HOPS)
    hl = jnp.arange(L_HOPS)
    send_r = ring[(r - hr) % N_DEV]
    recv_r = ring[(r - 1 - hr) % N_DEV]
    send_l = ring[(r + hl) % N_DEV]
    recv_l = ring[(r + 1 + hl) % N_DEV]

    meta = jnp.concatenate(
        [right[None], left[None], send_r, recv_r, send_l, recv_l]
    ).astype(jnp.int32)

    def body(meta_ref, x_ref, w_ref, sx_ref, sw_ref, out_ref,
             xfull_ref, sems_sr, sems_rr, sems_sl, sems_rl):
        rt = meta_ref[0]
        lt = meta_ref[1]

        barrier_sem = pltpu.get_barrier_semaphore()
        for nbr in (lt, rt):
            pl.semaphore_signal(
                barrier_sem, inc=1,
                device_id=(nbr,), device_id_type=pl.DeviceIdType.MESH,
            )
        pl.semaphore_wait(barrier_sem, 2)

        my_rows = meta_ref[2] * m_per
        xfull_ref[pl.ds(my_rows, m_per), :] = x_ref[...]

        def sub_slab(origin, s):
            return xfull_ref.at[pl.ds(origin * m_per + s * sub_m, sub_m), :]

        def mk(origin, h, s, to_right):
            return pltpu.make_async_remote_copy(
                src_ref=sub_slab(origin, s), dst_ref=sub_slab(origin, s),
                send_sem=(sems_sr if to_right else sems_sl).at[h, s],
                recv_sem=(sems_rr if to_right else sems_rl).at[h, s],
                device_id=(rt if to_right else lt,),
                device_id_type=pl.DeviceIdType.MESH,
            )

        senders = []
        for s in range(SUBS):
            d = mk(meta_ref[2], 0, s, True)
            d.start()
            senders.append(d)
        for s in range(SUBS):
            d = mk(meta_ref[34], 0, s, False)
            d.start()
            senders.append(d)

        for h in range(1, R_HOPS):
            for s in range(SUBS):
                mk(meta_ref[18 + h - 1], h - 1, s, True).wait_recv()
                d = mk(meta_ref[2 + h], h, s, True)
                d.start()
                senders.append(d)
                if h < L_HOPS:
                    mk(meta_ref[49 + h - 1], h - 1, s, False).wait_recv()
                    d = mk(meta_ref[34 + h], h, s, False)
                    d.start()
                    senders.append(d)

        for s in range(SUBS):
            mk(meta_ref[18 + R_HOPS - 1], R_HOPS - 1, s, True).wait_recv()
        for s in range(SUBS):
            mk(meta_ref[49 + L_HOPS - 1], L_HOPS - 1, s, False).wait_recv()

        for d in senders:
            d.wait_send()

        acc = lax.dot_general(
            xfull_ref[...], w_ref[...],
            (((1,), (0,)), ((), ())),
            preferred_element_type=jnp.int32,
        )
        scale = sx_ref[0] * sw_ref[0]
        out_ref[...] = jnp.maximum(acc.astype(jnp.float32) * scale, 0.0)

    return pl.pallas_call(
        body,
        out_shape=jax.ShapeDtypeStruct((N_DEV * m_per, n_per), jnp.float32),
        in_specs=[
            pl.BlockSpec(memory_space=pltpu.SMEM),
            pl.BlockSpec(memory_space=pltpu.VMEM),
            pl.BlockSpec(memory_space=pltpu.VMEM),
            pl.BlockSpec(memory_space=pltpu.SMEM),
            pl.BlockSpec(memory_space=pltpu.SMEM),
        ],
        out_specs=pl.BlockSpec(memory_space=pltpu.VMEM),
        scratch_shapes=[
            pltpu.VMEM((N_DEV * m_per, k), jnp.int8),
            pltpu.SemaphoreType.DMA((R_HOPS, SUBS)),
            pltpu.SemaphoreType.DMA((R_HOPS, SUBS)),
            pltpu.SemaphoreType.DMA((L_HOPS, SUBS)),
            pltpu.SemaphoreType.DMA((L_HOPS, SUBS)),
        ],
        compiler_params=pltpu.CompilerParams(collective_id=0),
    )(meta, x, w_mat, scale_x, scale_w)


# device time: 107574 ns/iter; 2.3152x vs baseline; 1.0910x over previous
import numpy as np

import jax
import jax.numpy as jnp
from jax import lax
from jax.experimental import pallas as pl
from jax.experimental.pallas import tpu as pltpu

N_DEV = 32
R_HOPS = 16
L_HOPS = 15
SUBS = 4

_ring_cache = None


def _hamiltonian_ring():
    global _ring_cache
    if _ring_cache is not None:
        return _ring_cache
    import distributed_mesh_v7x as dm

    mesh = dm.get_mesh("i", world_size=N_DEV)
    coords = [tuple(d.coords) for d in mesh.devices.flat]
    coord_to_logical = {c: j for j, c in enumerate(coords)}
    p = [(0, 0), (1, 0), (2, 0), (3, 0), (3, 1), (2, 1), (1, 1), (0, 1),
         (0, 2), (1, 2), (2, 2), (3, 2), (3, 3), (2, 3), (1, 3), (0, 3)]
    cycle = [(0, y, z) for (y, z) in p] + [(1, y, z) for (y, z) in reversed(p)]
    ring = np.array([coord_to_logical[c] for c in cycle], dtype=np.int32)
    inv = np.empty(N_DEV, dtype=np.int32)
    inv[ring] = np.arange(N_DEV, dtype=np.int32)
    _ring_cache = (ring, inv)
    return _ring_cache


def kernel(x, w_mat, scale_x, scale_w):
    m_per, k = x.shape
    n_per = w_mat.shape[1]
    sub_m = m_per // SUBS

    ring_np, inv_np = _hamiltonian_ring()
    ring = jnp.asarray(ring_np)
    my = lax.axis_index("i")
    r = jnp.asarray(inv_np)[my]
    right = ring[(r + 1) % N_DEV]
    left = ring[(r - 1) % N_DEV]

    hr = jnp.arange(R_HOPS)
    hl = jnp.arange(L_HOPS)
    send_r = ring[(r - hr) % N_DEV]
    recv_r = ring[(r - 1 - hr) % N_DEV]
    send_l = ring[(r + hl) % N_DEV]
    recv_l = ring[(r + 1 + hl) % N_DEV]

    meta = jnp.concatenate(
        [right[None], left[None], send_r, recv_r, send_l, recv_l]
    ).astype(jnp.int32)

    def body(meta_ref, x_ref, w_ref, sx_ref, sw_ref, out_ref,
             xfull_ref, sems_sr, sems_rr, sems_sl, sems_rl):
        rt = meta_ref[0]
        lt = meta_ref[1]

        barrier_sem = pltpu.get_barrier_semaphore()
        for nbr in (lt, rt):
            pl.semaphore_signal(
                barrier_sem, inc=1,
                device_id=(nbr,), device_id_type=pl.DeviceIdType.MESH,
            )
        pl.semaphore_wait(barrier_sem, 2)

        scale = sx_ref[0] * sw_ref[0]

        def gemm_chunk(origin):
            rows = origin * m_per
            acc = lax.dot_general(
                xfull_ref[pl.ds(rows, m_per), :], w_ref[...],
                (((1,), (0,)), ((), ())),
                preferred_element_type=jnp.int32,
            )
            out_ref[pl.ds(rows, m_per), :] = jnp.maximum(
                acc.astype(jnp.float32) * scale, 0.0
            )

        my_rows = meta_ref[2] * m_per
        xfull_ref[pl.ds(my_rows, m_per), :] = x_ref[...]

        def sub_slab(origin, s):
            return xfull_ref.at[pl.ds(origin * m_per + s * sub_m, sub_m), :]

        def mk(origin, h, s, to_right):
            return pltpu.make_async_remote_copy(
                src_ref=sub_slab(origin, s), dst_ref=sub_slab(origin, s),
                send_sem=(sems_sr if to_right else sems_sl).at[h, s],
                recv_sem=(sems_rr if to_right else sems_rl).at[h, s],
                device_id=(rt if to_right else lt,),
                device_id_type=pl.DeviceIdType.MESH,
            )

        senders = []
        for s in range(SUBS):
            d = mk(meta_ref[2], 0, s, True)
            d.start()
            senders.append(d)
        for s in range(SUBS):
            d = mk(meta_ref[34], 0, s, False)
            d.start()
            senders.append(d)

        gemm_chunk(meta_ref[2])

        for h in range(1, R_HOPS):
            for s in range(SUBS):
                mk(meta_ref[18 + h - 1], h - 1, s, True).wait_recv()
                d = mk(meta_ref[2 + h], h, s, True)
                d.start()
                senders.append(d)
                if h - 1 < L_HOPS:
                    mk(meta_ref[49 + h - 1], h - 1, s, False).wait_recv()
                    if h < L_HOPS:
                        d = mk(meta_ref[34 + h], h, s, False)
                        d.start()
                        senders.append(d)
            gemm_chunk(meta_ref[18 + h - 1])
            if h - 1 < L_HOPS:
                gemm_chunk(meta_ref[49 + h - 1])

        for s in range(SUBS):
            mk(meta_ref[18 + R_HOPS - 1], R_HOPS - 1, s, True).wait_recv()

        for d in senders:
            d.wait_send()

        gemm_chunk(meta_ref[18 + R_HOPS - 1])

    return pl.pallas_call(
        body,
        out_shape=jax.ShapeDtypeStruct((N_DEV * m_per, n_per), jnp.float32),
        in_specs=[
            pl.BlockSpec(memory_space=pltpu.SMEM),
            pl.BlockSpec(memory_space=pltpu.VMEM),
            pl.BlockSpec(memory_space=pltpu.VMEM),
            pl.BlockSpec(memory_space=pltpu.SMEM),
            pl.BlockSpec(memory_space=pltpu.SMEM),
        ],
        out_specs=pl.BlockSpec(memory_space=pltpu.VMEM),
        scratch_shapes=[
            pltpu.VMEM((N_DEV * m_per, k), jnp.int8),
            pltpu.SemaphoreType.DMA((R_HOPS, SUBS)),
            pltpu.SemaphoreType.DMA((R_HOPS, SUBS)),
            pltpu.SemaphoreType.DMA((L_HOPS, SUBS)),
            pltpu.SemaphoreType.DMA((L_HOPS, SUBS)),
        ],
        compiler_params=pltpu.CompilerParams(collective_id=0),
    )(meta, x, w_mat, scale_x, scale_w)


# device time: 106131 ns/iter; 2.3467x vs baseline; 1.0136x over previous
import numpy as np

import jax
import jax.numpy as jnp
from jax import lax
from jax.experimental import pallas as pl
from jax.experimental.pallas import tpu as pltpu

N_DEV = 32
HOPS = 16
SUBS = 4
R_ANTI = (0, 1)
L_ANTI = (2, 3)

_ring_cache = None


def _hamiltonian_ring():
    global _ring_cache
    if _ring_cache is not None:
        return _ring_cache
    import distributed_mesh_v7x as dm

    mesh = dm.get_mesh("i", world_size=N_DEV)
    coords = [tuple(d.coords) for d in mesh.devices.flat]
    coord_to_logical = {c: j for j, c in enumerate(coords)}
    p = [(0, 0), (1, 0), (2, 0), (3, 0), (3, 1), (2, 1), (1, 1), (0, 1),
         (0, 2), (1, 2), (2, 2), (3, 2), (3, 3), (2, 3), (1, 3), (0, 3)]
    cycle = [(0, y, z) for (y, z) in p] + [(1, y, z) for (y, z) in reversed(p)]
    ring = np.array([coord_to_logical[c] for c in cycle], dtype=np.int32)
    inv = np.empty(N_DEV, dtype=np.int32)
    inv[ring] = np.arange(N_DEV, dtype=np.int32)
    _ring_cache = (ring, inv)
    return _ring_cache


def _r_subs(h):
    return R_ANTI if h == HOPS - 1 else tuple(range(SUBS))


def _l_subs(h):
    return L_ANTI if h == HOPS - 1 else tuple(range(SUBS))


def kernel(x, w_mat, scale_x, scale_w):
    m_per, k = x.shape
    n_per = w_mat.shape[1]
    sub_m = m_per // SUBS

    ring_np, inv_np = _hamiltonian_ring()
    ring = jnp.asarray(ring_np)
    my = lax.axis_index("i")
    r = jnp.asarray(inv_np)[my]
    right = ring[(r + 1) % N_DEV]
    left = ring[(r - 1) % N_DEV]

    h16 = jnp.arange(HOPS)
    send_r = ring[(r - h16) % N_DEV]
    recv_r = ring[(r - 1 - h16) % N_DEV]
    send_l = ring[(r + h16) % N_DEV]
    recv_l = ring[(r + 1 + h16) % N_DEV]

    meta = jnp.concatenate(
        [right[None], left[None], send_r, recv_r, send_l, recv_l]
    ).astype(jnp.int32)

    def body(meta_ref, x_ref, w_ref, sx_ref, sw_ref, out_ref,
             xfull_ref, sems_sr, sems_rr, sems_sl, sems_rl):
        rt = meta_ref[0]
        lt = meta_ref[1]

        barrier_sem = pltpu.get_barrier_semaphore()
        for nbr in (lt, rt):
            pl.semaphore_signal(
                barrier_sem, inc=1,
                device_id=(nbr,), device_id_type=pl.DeviceIdType.MESH,
            )
        pl.semaphore_wait(barrier_sem, 2)

        scale = sx_ref[0] * sw_ref[0]

        def gemm_chunk(origin):
            rows = origin * m_per
            acc = lax.dot_general(
                xfull_ref[pl.ds(rows, m_per), :], w_ref[...],
                (((1,), (0,)), ((), ())),
                preferred_element_type=jnp.int32,
            )
            out_ref[pl.ds(rows, m_per), :] = jnp.maximum(
                acc.astype(jnp.float32) * scale, 0.0
            )

        my_rows = meta_ref[2] * m_per
        xfull_ref[pl.ds(my_rows, m_per), :] = x_ref[...]

        def sub_slab(origin, s):
            return xfull_ref.at[pl.ds(origin * m_per + s * sub_m, sub_m), :]

        def mk(origin, h, s, to_right):
            return pltpu.make_async_remote_copy(
                src_ref=sub_slab(origin, s), dst_ref=sub_slab(origin, s),
                send_sem=(sems_sr if to_right else sems_sl).at[h, s],
                recv_sem=(sems_rr if to_right else sems_rl).at[h, s],
                device_id=(rt if to_right else lt,),
                device_id_type=pl.DeviceIdType.MESH,
            )

        senders = []
        for s in range(SUBS):
            d = mk(meta_ref[2], 0, s, True)
            d.start()
            senders.append(d)
        for s in range(SUBS):
            d = mk(meta_ref[34], 0, s, False)
            d.start()
            senders.append(d)

        gemm_chunk(meta_ref[2])

        for h in range(1, HOPS):
            for s in range(SUBS):
                mk(meta_ref[18 + h - 1], h - 1, s, True).wait_recv()
                if s in _r_subs(h):
                    d = mk(meta_ref[2 + h], h, s, True)
                    d.start()
                    senders.append(d)
                mk(meta_ref[50 + h - 1], h - 1, s, False).wait_recv()
                if s in _l_subs(h):
                    d = mk(meta_ref[34 + h], h, s, False)
                    d.start()
                    senders.append(d)
            gemm_chunk(meta_ref[18 + h - 1])
            gemm_chunk(meta_ref[50 + h - 1])

        for s in R_ANTI:
            mk(meta_ref[18 + HOPS - 1], HOPS - 1, s, True).wait_recv()
        for s in L_ANTI:
            mk(meta_ref[50 + HOPS - 1], HOPS - 1, s, False).wait_recv()

        for d in senders:
            d.wait_send()

        gemm_chunk(meta_ref[18 + HOPS - 1])

    return pl.pallas_call(
        body,
        out_shape=jax.ShapeDtypeStruct((N_DEV * m_per, n_per), jnp.float32),
        in_specs=[
            pl.BlockSpec(memory_space=pltpu.SMEM),
            pl.BlockSpec(memory_space=pltpu.VMEM),
            pl.BlockSpec(memory_space=pltpu.VMEM),
            pl.BlockSpec(memory_space=pltpu.SMEM),
            pl.BlockSpec(memory_space=pltpu.SMEM),
        ],
        out_specs=pl.BlockSpec(memory_space=pltpu.VMEM),
        scratch_shapes=[
            pltpu.VMEM((N_DEV * m_per, k), jnp.int8),
            pltpu.SemaphoreType.DMA((HOPS, SUBS)),
            pltpu.SemaphoreType.DMA((HOPS, SUBS)),
            pltpu.SemaphoreType.DMA((HOPS, SUBS)),
            pltpu.SemaphoreType.DMA((HOPS, SUBS)),
        ],
        compiler_params=pltpu.CompilerParams(collective_id=0),
    )(meta, x, w_mat, scale_x, scale_w)
